# Initial kernel scaffold; baseline (speedup 1.0000x reference)
import jax
import jax.numpy as jnp
from jax.experimental import pallas as pl


def kernel(confidence, predicted_locations, labels, gt_locations):
    raise NotImplementedError("write your pallas kernel here")


# speedup vs baseline: 3.9056x; 3.9056x over previous
import jax
import jax.numpy as jnp
from jax.experimental import pallas as pl
from jax.experimental.pallas import tpu as pltpu

_NEG_RATIO = 3


def _body(conf_ref, locst_ref, gtt_ref, labc_ref, labr_ref, out_ref,
          acc_ref, cls_ref):
    b = pl.program_id(0)
    nb = pl.num_programs(0)
    P, C = conf_ref.shape[1], conf_ref.shape[2]

    @pl.when(b == 0)
    def _():
        acc_ref[0] = 0.0
        acc_ref[1] = 0.0
        acc_ref[2] = 0.0
        acc_ref[3] = 0.0

    x = conf_ref[0]
    lab = labc_ref[0]
    rowmax = jnp.max(x, axis=1, keepdims=True)
    lse = rowmax + jnp.log(jnp.sum(jnp.exp(x - rowmax), axis=1,
                                   keepdims=True))
    loss = lse - x[:, 0:1]
    ci = jax.lax.broadcasted_iota(jnp.int32, (P, C), 1)
    x_at_lab = jnp.sum(jnp.where(ci == lab, x, 0.0), axis=1, keepdims=True)
    ce = lse - x_at_lab
    pos = lab > 0
    npos = jnp.sum(pos.astype(jnp.int32))

    dl = locst_ref[0] - gtt_ref[0]
    ad = jnp.abs(dl)
    e = jnp.where(ad < 1.0, 0.5 * dl * dl, ad - 0.5)
    rs = jnp.sum(e, axis=0, keepdims=True)
    pos_row = labr_ref[0] > 0
    sl1 = jnp.sum(jnp.where(pos_row, rs, 0.0))

    @pl.when(4 * npos >= P)
    def _():
        cls_ref[0] = jnp.sum(ce)

    @pl.when(4 * npos < P)
    def _():
        neg_inf = jnp.float32(-jnp.inf)
        loss_hn = jnp.where(pos, neg_inf, loss)
        bits = jax.lax.bitcast_convert_type(loss_hn, jnp.int32)
        key = jnp.where(bits >= 0, bits, bits ^ jnp.int32(0x7FFFFFFF))
        K = jnp.where(npos > 0, _NEG_RATIO * npos, _NEG_RATIO)
        int_min = jnp.int32(-2147483648)

        def bit_step(i, t):
            sb = 31 - i
            trial = jnp.where(sb == 31, t ^ int_min,
                              t | (jnp.int32(1) << sb))
            cnt = jnp.sum((key >= trial).astype(jnp.int32))
            return jnp.where(cnt >= K, trial, t)

        t = jax.lax.fori_loop(0, 32, bit_step, int_min)
        gt_m = key > t
        eq = key == t
        m = K - jnp.sum(gt_m.astype(jnp.int32))
        idxv = jax.lax.broadcasted_iota(jnp.int32, (P, 1), 0)

        def jbit_step(i, j):
            trial = j | (jnp.int32(1) << (13 - i))
            c = jnp.sum((eq & (idxv < trial)).astype(jnp.int32))
            return jnp.where(c < m, trial, j)

        jstar = jax.lax.fori_loop(0, 14, jbit_step, jnp.int32(0))
        sel = gt_m | (eq & (idxv <= jstar))
        cls_ref[0] = jnp.sum(jnp.where(pos | sel, ce, 0.0))

    acc_ref[0] += sl1
    acc_ref[1] += npos.astype(jnp.float32)
    acc_ref[2] += cls_ref[0]
    acc_ref[3] += jnp.where(npos > 0, 0.0, 3.0)

    @pl.when(b == nb - 1)
    def _():
        tp = acc_ref[1]
        l1 = acc_ref[0] / jnp.maximum(tp, 1.0)
        l2 = acc_ref[2] / jnp.maximum(tp + acc_ref[3], 1.0)
        li = jax.lax.broadcasted_iota(jnp.int32, (1, 128), 1)
        out_ref[...] = jnp.where(li == 0, l1, jnp.where(li == 1, l2, 0.0))


def kernel(confidence, predicted_locations, labels, gt_locations):
    B, P, C = confidence.shape
    locs_t = jnp.transpose(predicted_locations, (0, 2, 1))
    gt_t = jnp.transpose(gt_locations, (0, 2, 1))
    lab_col = labels[..., None]
    lab_row = labels[:, None, :]

    out = pl.pallas_call(
        _body,
        grid=(B,),
        in_specs=[
            pl.BlockSpec((1, P, C), lambda b: (b, 0, 0)),
            pl.BlockSpec((1, 4, P), lambda b: (b, 0, 0)),
            pl.BlockSpec((1, 4, P), lambda b: (b, 0, 0)),
            pl.BlockSpec((1, P, 1), lambda b: (b, 0, 0)),
            pl.BlockSpec((1, 1, P), lambda b: (b, 0, 0)),
        ],
        out_specs=pl.BlockSpec((1, 128), lambda b: (0, 0)),
        out_shape=jax.ShapeDtypeStruct((1, 128), jnp.float32),
        scratch_shapes=[
            pltpu.SMEM((8,), jnp.float32),
            pltpu.SMEM((1,), jnp.float32),
        ],
    )(confidence, locs_t, gt_t, lab_col, lab_row)
    return (out[0, 0], out[0, 1])


# speedup vs baseline: 8.0530x; 2.0619x over previous
import jax
import jax.numpy as jnp
from jax.experimental import pallas as pl
from jax.experimental.pallas import tpu as pltpu

_NEG_RATIO = 3


def _body(conf_ref, locst_ref, gtt_ref, labr_ref, out_ref, acc_ref, cls_ref):
    b = pl.program_id(0)
    nb = pl.num_programs(0)
    P, C = conf_ref.shape[1], conf_ref.shape[2]

    @pl.when(b == 0)
    def _():
        acc_ref[0] = 0.0
        acc_ref[1] = 0.0
        acc_ref[2] = 0.0
        acc_ref[3] = 0.0

    xt = jnp.swapaxes(conf_ref[0], 0, 1)
    lab = labr_ref[0]
    colmax = jnp.max(xt, axis=0, keepdims=True)
    s = jnp.sum(jnp.exp(xt - colmax), axis=0, keepdims=True)
    lse = colmax + jnp.log(s)
    ci = jax.lax.broadcasted_iota(jnp.int32, (C, P), 0)
    x_at_lab = jnp.sum(jnp.where(ci == lab, xt, 0.0), axis=0, keepdims=True)
    ce = lse - x_at_lab
    pos = lab > 0
    npos = jnp.sum(pos.astype(jnp.int32))

    dl = locst_ref[0] - gtt_ref[0]
    ad = jnp.abs(dl)
    e = jnp.where(ad < 1.0, 0.5 * dl * dl, ad - 0.5)
    rs = jnp.sum(e, axis=0, keepdims=True)
    sl1 = jnp.sum(jnp.where(pos, rs, 0.0))

    @pl.when(4 * npos >= P)
    def _():
        cls_ref[0] = jnp.sum(ce)

    @pl.when(4 * npos < P)
    def _():
        loss = lse - xt[0:1, :]
        neg_inf = jnp.float32(-jnp.inf)
        loss_hn = jnp.where(pos, neg_inf, loss)
        bits = jax.lax.bitcast_convert_type(loss_hn, jnp.int32)
        key = jnp.where(bits >= 0, bits, bits ^ jnp.int32(0x7FFFFFFF))
        K = jnp.where(npos > 0, _NEG_RATIO * npos, _NEG_RATIO)
        int_min = jnp.int32(-2147483648)

        def bit_step(i, t):
            sb = 31 - i
            trial = jnp.where(sb == 31, t ^ int_min,
                              t | (jnp.int32(1) << sb))
            cnt = jnp.sum((key >= trial).astype(jnp.int32))
            return jnp.where(cnt >= K, trial, t)

        t = jax.lax.fori_loop(0, 32, bit_step, int_min)
        gt_m = key > t
        eq = key == t
        m = K - jnp.sum(gt_m.astype(jnp.int32))
        idxv = jax.lax.broadcasted_iota(jnp.int32, (1, P), 1)

        def jbit_step(i, j):
            trial = j | (jnp.int32(1) << (13 - i))
            c = jnp.sum((eq & (idxv < trial)).astype(jnp.int32))
            return jnp.where(c < m, trial, j)

        jstar = jax.lax.fori_loop(0, 14, jbit_step, jnp.int32(0))
        sel = gt_m | (eq & (idxv <= jstar))
        cls_ref[0] = jnp.sum(jnp.where(pos | sel, ce, 0.0))

    acc_ref[0] += sl1
    acc_ref[1] += npos.astype(jnp.float32)
    acc_ref[2] += cls_ref[0]
    acc_ref[3] += jnp.where(npos > 0, 0.0, 3.0)

    @pl.when(b == nb - 1)
    def _():
        tp = acc_ref[1]
        l1 = acc_ref[0] / jnp.maximum(tp, 1.0)
        l2 = acc_ref[2] / jnp.maximum(tp + acc_ref[3], 1.0)
        li = jax.lax.broadcasted_iota(jnp.int32, (1, 128), 1)
        out_ref[...] = jnp.where(li == 0, l1, jnp.where(li == 1, l2, 0.0))


def kernel(confidence, predicted_locations, labels, gt_locations):
    B, P, C = confidence.shape
    locs_t = jnp.transpose(predicted_locations, (0, 2, 1))
    gt_t = jnp.transpose(gt_locations, (0, 2, 1))
    lab_row = labels[:, None, :]

    out = pl.pallas_call(
        _body,
        grid=(B,),
        in_specs=[
            pl.BlockSpec((1, P, C), lambda b: (b, 0, 0)),
            pl.BlockSpec((1, 4, P), lambda b: (b, 0, 0)),
            pl.BlockSpec((1, 4, P), lambda b: (b, 0, 0)),
            pl.BlockSpec((1, 1, P), lambda b: (b, 0, 0)),
        ],
        out_specs=pl.BlockSpec((1, 128), lambda b: (0, 0)),
        out_shape=jax.ShapeDtypeStruct((1, 128), jnp.float32),
        scratch_shapes=[
            pltpu.SMEM((8,), jnp.float32),
            pltpu.SMEM((1,), jnp.float32),
        ],
    )(confidence, locs_t, gt_t, lab_row)
    return (out[0, 0], out[0, 1])
